# baseline (speedup 1.0000x reference)
import functools

import jax
import jax.numpy as jnp
from jax import lax
from jax.experimental import pallas as pl
from jax.experimental.pallas import tpu as pltpu
from jax.experimental.pallas import tpu_sc as plsc

N_COLS = 32768
N_ROWS = 128
N_IDX = 16384

_info = plsc.get_sparse_core_info()
_NC, _NS, _L = _info.num_cores, _info.num_subcores, _info.num_lanes
_NW = _NC * _NS
_IDX_PER_W = N_IDX // _NW
_IDX_CHUNK = 128
_IDX_ROWS = _IDX_PER_W // _IDX_CHUNK
_SEG = N_COLS // _NS


def _sc_body(idx_hbm, cnt_hbm, idx_v, zeros_v, ones_v, shared):
    cid = lax.axis_index("c")
    sid = lax.axis_index("s")
    wid = sid * _NC + cid

    z16 = jnp.zeros((_L,), jnp.float32)
    o16 = jnp.ones((_L,), jnp.float32)
    for i in range(_SEG // _L):
        zeros_v[pl.ds(i * _L, _L)] = z16
    for i in range(_IDX_CHUNK // _L):
        ones_v[pl.ds(i * _L, _L)] = o16

    pltpu.sync_copy(zeros_v, shared.at[pl.ds(sid * _SEG, _SEG)])
    plsc.subcore_barrier()

    pltpu.sync_copy(idx_hbm.at[wid], idx_v)
    for j in range(_IDX_ROWS):
        pltpu.sync_copy(ones_v, shared.at[idx_v.at[j]], add=True)
    plsc.subcore_barrier()

    pltpu.sync_copy(shared.at[pl.ds(sid * _SEG, _SEG)],
                    cnt_hbm.at[cid, pl.ds(sid * _SEG, _SEG)])


_sc_count = functools.partial(
    pl.kernel,
    mesh=plsc.VectorSubcoreMesh(core_axis_name="c", subcore_axis_name="s"),
    out_type=jax.ShapeDtypeStruct((_NC, N_COLS), jnp.float32),
    scratch_types=[
        pltpu.VMEM((_IDX_ROWS, _IDX_CHUNK), jnp.int32),
        pltpu.VMEM((_SEG,), jnp.float32),
        pltpu.VMEM((_IDX_CHUNK,), jnp.float32),
        pltpu.VMEM_SHARED((N_COLS,), jnp.float32),
    ],
)(_sc_body)


_BN = 1024


def _tc_body(x_ref, c_ref, o_ref):
    keep = (c_ref[0:1, :] + c_ref[1:2, :]) > 0.0
    o_ref[...] = jnp.where(keep, x_ref[...], 0.0)


_tc_mult = pl.pallas_call(
    _tc_body,
    grid=(N_COLS // _BN,),
    in_specs=[
        pl.BlockSpec((N_ROWS, _BN), lambda i: (0, i)),
        pl.BlockSpec((2, _BN), lambda i: (0, i)),
    ],
    out_specs=pl.BlockSpec((N_ROWS, _BN), lambda i: (0, i)),
    out_shape=jax.ShapeDtypeStruct((N_ROWS, N_COLS), jnp.float32),
)


def kernel(x, retain_idx):
    idx3 = retain_idx.reshape(_NW, _IDX_ROWS, _IDX_CHUNK)
    cnt = _sc_count(idx3)
    return _tc_mult(x, cnt)


# speedup vs baseline: 1.3497x; 1.3497x over previous
import functools

import jax
import jax.numpy as jnp
from jax import lax
from jax.experimental import pallas as pl
from jax.experimental.pallas import tpu as pltpu
from jax.experimental.pallas import tpu_sc as plsc

N_COLS = 32768
N_ROWS = 128
N_IDX = 16384

_info = plsc.get_sparse_core_info()
_NC, _NS, _L = _info.num_cores, _info.num_subcores, _info.num_lanes
_NW = _NC * _NS
_IDX_PER_W = N_IDX // _NW
_IDX_CHUNK = 128
_IDX_ROWS = _IDX_PER_W // _IDX_CHUNK
_SEG = N_COLS // _NS


def _sc_body(idx_hbm, cnt_hbm, idx_v, zeros_v, ones_v, shared):
    cid = lax.axis_index("c")
    sid = lax.axis_index("s")
    wid = sid * _NC + cid

    z16 = jnp.zeros((_L,), jnp.float32)
    o16 = jnp.ones((_L,), jnp.float32)
    for i in range(_SEG // _L):
        zeros_v[pl.ds(i * _L, _L)] = z16
    for i in range(_IDX_CHUNK // _L):
        ones_v[pl.ds(i * _L, _L)] = o16

    pltpu.sync_copy(zeros_v, shared.at[pl.ds(sid * _SEG, _SEG)])
    plsc.subcore_barrier()

    pltpu.sync_copy(idx_hbm.at[wid], idx_v)
    for j in range(_IDX_ROWS):
        pltpu.sync_copy(ones_v, shared.at[idx_v.at[j]], add=True)
    plsc.subcore_barrier()

    pltpu.sync_copy(shared.at[pl.ds(sid * _SEG, _SEG)],
                    cnt_hbm.at[cid, pl.ds(sid * _SEG, _SEG)])


_sc_count = functools.partial(
    pl.kernel,
    mesh=plsc.VectorSubcoreMesh(core_axis_name="c", subcore_axis_name="s"),
    out_type=jax.ShapeDtypeStruct((_NC, N_COLS), jnp.float32),
    scratch_types=[
        pltpu.VMEM((_IDX_ROWS, _IDX_CHUNK), jnp.int32),
        pltpu.VMEM((_SEG,), jnp.float32),
        pltpu.VMEM((_IDX_CHUNK,), jnp.float32),
        pltpu.VMEM_SHARED((N_COLS,), jnp.float32),
    ],
)(_sc_body)


_BN = 8192


def _tc_body(x_ref, c_ref, o_ref):
    keep = (c_ref[0:1, :] + c_ref[1:2, :]) > 0.0
    o_ref[...] = jnp.where(keep, x_ref[...], 0.0)


_tc_mult = pl.pallas_call(
    _tc_body,
    grid=(N_COLS // _BN,),
    in_specs=[
        pl.BlockSpec((N_ROWS, _BN), lambda i: (0, i)),
        pl.BlockSpec((2, _BN), lambda i: (0, i)),
    ],
    out_specs=pl.BlockSpec((N_ROWS, _BN), lambda i: (0, i)),
    out_shape=jax.ShapeDtypeStruct((N_ROWS, N_COLS), jnp.float32),
)


def kernel(x, retain_idx):
    idx3 = retain_idx.reshape(_NW, _IDX_ROWS, _IDX_CHUNK)
    cnt = _sc_count(idx3)
    return _tc_mult(x, cnt)
